# baseline (speedup 1.0000x reference)
import functools

import jax
import jax.numpy as jnp
from jax import lax
from jax.experimental import pallas as pl
from jax.experimental.pallas import tpu as pltpu
from jax.experimental.pallas import tpu_sc as plsc

_B = 16384
_D = 32
_LANES = 16
_PACK = 4
_ROWPAD = _PACK * _D

_NC = 2
_NS = 16
_NW = _NC * _NS
_BPW = _B // _NW
_NCHUNK = 2
_CPW = _BPW // _NCHUNK


def _sc_body(uidx_hbm, iidx_hbm, utab_hbm, itab_hbm, out_hbm,
             uidx_v, iidx_v, uq_v, iq_v, uoff_v, ioff_v,
             urows_v, irows_v, out_v, sem):
    wid = lax.axis_index("s") * _NC + lax.axis_index("c")
    base = wid * _BPW

    pltpu.sync_copy(uidx_hbm.at[pl.ds(base, _BPW)], uidx_v)
    pltpu.sync_copy(iidx_hbm.at[pl.ds(base, _BPW)], iidx_v)

    def split(g, carry):
        gbase = g * _LANES
        xu = uidx_v[pl.ds(gbase, _LANES)]
        xi = iidx_v[pl.ds(gbase, _LANES)]
        uq_v[pl.ds(gbase, _LANES)] = lax.shift_right_logical(xu, 2)
        iq_v[pl.ds(gbase, _LANES)] = lax.shift_right_logical(xi, 2)
        uoff_v[pl.ds(gbase, _LANES)] = (xu & 3) * _D
        ioff_v[pl.ds(gbase, _LANES)] = (xi & 3) * _D
        return carry

    lax.fori_loop(0, _BPW // _LANES, split, 0)

    lane_iota = lax.iota(jnp.int32, _LANES)

    for c in range(_NCHUNK):
        cu = pltpu.async_copy(
            utab_hbm.at[uq_v.at[pl.ds(c * _CPW, _CPW)]], urows_v, sem)
        ci = pltpu.async_copy(
            itab_hbm.at[iq_v.at[pl.ds(c * _CPW, _CPW)]], irows_v, sem)
        cu.wait()
        ci.wait()

        def group(g, carry):
            gbase = g * _LANES
            pou = uoff_v[pl.ds(c * _CPW + gbase, _LANES)]
            poi = ioff_v[pl.ds(c * _CPW + gbase, _LANES)]
            acc = jnp.zeros((_LANES,), jnp.float32)
            for j in range(_LANES):
                r = gbase + j
                po = pou[j]
                qo = poi[j]
                u0 = urows_v[r, pl.ds(po, _LANES)]
                u1 = urows_v[r, pl.ds(po + _LANES, _LANES)]
                i0 = irows_v[r, pl.ds(qo, _LANES)]
                i1 = irows_v[r, pl.ds(qo + _LANES, _LANES)]
                s = jnp.sum(u0 * i0 + u1 * i1)
                acc = jnp.where(lane_iota == j, s, acc)
            out_v[pl.ds(c * _CPW + gbase, _LANES)] = (
                1.0 / (1.0 + jnp.exp(-acc)))
            return carry

        lax.fori_loop(0, _CPW // _LANES, group, 0)

    pltpu.sync_copy(out_v, out_hbm.at[pl.ds(base, _BPW)])


@jax.jit
def _mf_forward(user_indices, item_indices, user_table, item_table):
    mesh = plsc.VectorSubcoreMesh(core_axis_name="c", subcore_axis_name="s")
    run = functools.partial(
        pl.kernel,
        mesh=mesh,
        compiler_params=pltpu.CompilerParams(
            needs_layout_passes=False, use_tc_tiling_on_sc=False
        ),
        out_type=jax.ShapeDtypeStruct((_B,), jnp.float32),
        scratch_types=[
            pltpu.VMEM((_BPW,), jnp.int32),
            pltpu.VMEM((_BPW,), jnp.int32),
            pltpu.VMEM((_BPW,), jnp.int32),
            pltpu.VMEM((_BPW,), jnp.int32),
            pltpu.VMEM((_BPW,), jnp.int32),
            pltpu.VMEM((_BPW,), jnp.int32),
            pltpu.VMEM((_CPW, _ROWPAD), jnp.float32),
            pltpu.VMEM((_CPW, _ROWPAD), jnp.float32),
            pltpu.VMEM((_BPW,), jnp.float32),
            pltpu.SemaphoreType.DMA,
        ],
    )(_sc_body)
    return run(user_indices, item_indices, user_table, item_table)


def kernel(user_indices, item_indices, user_table, item_table):
    n_users, d = user_table.shape
    n_items, _ = item_table.shape
    ut, it = lax.optimization_barrier(
        (user_table.reshape(n_users * d // _ROWPAD, _ROWPAD),
         item_table.reshape(n_items * d // _ROWPAD, _ROWPAD)))
    return _mf_forward(
        user_indices.astype(jnp.int32),
        item_indices.astype(jnp.int32),
        ut,
        it,
    )
